# Initial kernel scaffold; baseline (speedup 1.0000x reference)
import jax
import jax.numpy as jnp
from jax.experimental import pallas as pl


def kernel(x1, edge_index1, x2, edge_index2, W, b):
    raise NotImplementedError("write your pallas kernel here")


# speedup vs baseline: 1.2755x; 1.2755x over previous
import jax
import jax.numpy as jnp
from jax import lax
from jax.experimental import pallas as pl
from jax.experimental.pallas import tpu as pltpu
from jax.experimental.pallas import tpu_sc as plsc

N = 10000
E = 320000
D = 128
NSUB = 16
RPT = 640
NPAD = NSUB * RPT
CHUNK = 4000
NVREG = CHUNK // 16
GP = 64
FB = 64
BLK = 512
PKSH = 15
PKMASK = (1 << PKSH) - 1
TRASH = RPT << PKSH
NEG_INF = float("-inf")


def _precompute_body(x1_ref, x2_ref, w_ref, b_ref, p_ref, q_ref):
    wt = w_ref[0:D, :]
    wb = w_ref[D : 2 * D, :]
    x1 = x1_ref[...]
    x2 = x2_ref[...]
    p_ref[...] = (
        jnp.dot(x1, wt - wb, preferred_element_type=jnp.float32) + b_ref[...]
    )
    q_ref[0] = jnp.dot(x1, wb, preferred_element_type=jnp.float32)
    q_ref[1] = jnp.dot(x2, wb, preferred_element_type=jnp.float32)


_precompute = pl.pallas_call(
    _precompute_body,
    grid=(NPAD // BLK,),
    in_specs=[
        pl.BlockSpec((BLK, D), lambda i: (i, 0)),
        pl.BlockSpec((BLK, D), lambda i: (i, 0)),
        pl.BlockSpec((2 * D, D), lambda i: (0, 0)),
        pl.BlockSpec((1, D), lambda i: (0, 0)),
    ],
    out_specs=[
        pl.BlockSpec((BLK, D), lambda i: (i, 0)),
        pl.BlockSpec((2, BLK, D), lambda i: (0, i, 0)),
    ],
    out_shape=[
        jax.ShapeDtypeStruct((NPAD, D), jnp.float32),
        jax.ShapeDtypeStruct((2, NPAD, D), jnp.float32),
    ],
)


def _sc_body(ei_ref, q_ref, p_ref, out_ref,
             rowbuf, colbuf, spk, scol, g_buf, f_buf, m_buf, smem, sem):
    c = lax.axis_index("c")
    s = lax.axis_index("s")
    lo = s * RPT

    lo_v = jnp.broadcast_to(lo, (16,)).astype(jnp.int32)
    hi_v = jnp.broadcast_to(lo + RPT, (16,)).astype(jnp.int32)
    qb_v = jnp.broadcast_to(c * NPAD, (16,)).astype(jnp.int32)
    trash_v = jnp.full((16,), TRASH, jnp.int32)
    lsplats = [
        jnp.full((16,), l, jnp.int32) for l in range(16)
    ]

    def init_m(i, _):
        for k in range(D // 16):
            m_buf[i, pl.ds(k * 16, 16)] = jnp.full((16,), NEG_INF, jnp.float32)
        return 0

    lax.fori_loop(0, RPT + 1, init_m, 0)

    def do_chunk(ch, _):
        off = ch * CHUNK
        pltpu.sync_copy(ei_ref.at[pl.ds(2 * c * E + off, CHUNK)], rowbuf)
        pltpu.sync_copy(ei_ref.at[pl.ds((2 * c + 1) * E + off, CHUNK)], colbuf)

        smem[0] = 0

        def scan_v(v, _):
            r = rowbuf[pl.ds(v * 16, 16)]
            cl = colbuf[pl.ds(v * 16, 16)]
            pk = ((r - lo_v) << PKSH) | (cl + qb_v)
            for l in range(16):
                rl = r[l]
                hit = (rl >= lo) & (rl < lo + RPT)

                @pl.when(hit)
                def _():
                    cnt = smem[0]
                    spk[pl.ds(cnt, 16)] = jnp.take(pk, lsplats[l])
                    smem[0] = cnt + 1

            return 0

        lax.fori_loop(0, NVREG, scan_v, 0)
        n_match = smem[0]

        for t in range(GP // 16 * 2):
            spk[pl.ds(n_match + t * 16, 16)] = trash_v

        def unpack(u, _):
            pkv = spk[pl.ds(u * 16, 16)]
            scol[pl.ds(u * 16, 16)] = pkv & PKMASK
            return 0

        lax.fori_loop(0, (n_match + 15) // 16 + GP // 16, unpack, 0)

        npieces = (n_match + GP - 1) // GP

        def do_piece(p, _):
            pltpu.async_copy(
                q_ref.at[scol.at[pl.ds(p * GP, GP)]], g_buf, sem
            ).wait()
            for t in range(GP // 16):
                w = spk[pl.ds(p * GP + t * 16, 16)]
                for l in range(16):
                    d = w[l] >> PKSH
                    for k in range(D // 16):
                        sl = pl.ds(k * 16, 16)
                        m_buf[d, sl] = jnp.maximum(
                            m_buf[d, sl], g_buf[t * 16 + l, sl]
                        )
            return 0

        lax.fori_loop(0, npieces, do_piece, 0)
        return 0

    lax.fori_loop(0, E // CHUNK, do_chunk, 0)

    def fin(t, _):
        base = lo + t * FB
        pltpu.sync_copy(p_ref.at[pl.ds(base, FB)], f_buf)

        def fr(r, _):
            for k in range(D // 16):
                sl = pl.ds(k * 16, 16)
                f_buf[r, sl] = jnp.maximum(
                    f_buf[r, sl] + m_buf[t * FB + r, sl], 0.0
                )
            return 0

        lax.fori_loop(0, FB, fr, 0)
        pltpu.sync_copy(f_buf, out_ref.at[pl.ds(c * NPAD + base, FB)])
        return 0

    lax.fori_loop(0, RPT // FB, fin, 0)


_sc_run = pl.kernel(
    _sc_body,
    out_type=jax.ShapeDtypeStruct((2 * NPAD, D), jnp.float32),
    mesh=plsc.VectorSubcoreMesh(core_axis_name="c", subcore_axis_name="s"),
    scratch_types=[
        pltpu.VMEM((CHUNK,), jnp.int32),
        pltpu.VMEM((CHUNK,), jnp.int32),
        pltpu.VMEM((CHUNK + 2 * GP,), jnp.int32),
        pltpu.VMEM((CHUNK + 2 * GP,), jnp.int32),
        pltpu.VMEM((GP, D), jnp.float32),
        pltpu.VMEM((FB, D), jnp.float32),
        pltpu.VMEM((RPT + 1, D), jnp.float32),
        pltpu.SMEM((8,), jnp.int32),
        pltpu.SemaphoreType.DMA,
    ],
)


@jax.jit
def kernel(x1, edge_index1, x2, edge_index2, W, b):
    x1p = jnp.pad(x1, ((0, NPAD - N), (0, 0)))
    x2p = jnp.pad(x2, ((0, NPAD - N), (0, 0)))
    p, qs = _precompute(x1p, x2p, W, b.reshape(1, D))
    qflat = qs.reshape(2 * NPAD, D)
    ei4 = jnp.concatenate([edge_index1, edge_index2], axis=0).reshape(4 * E)
    outs = _sc_run(ei4, qflat, p)
    return jnp.concatenate([outs[:N], outs[NPAD : NPAD + N]], axis=-1)


# speedup vs baseline: 1.3882x; 1.0884x over previous
import jax
import jax.numpy as jnp
from jax import lax
from jax.experimental import pallas as pl
from jax.experimental.pallas import tpu as pltpu
from jax.experimental.pallas import tpu_sc as plsc

N = 10000
E = 320000
D = 128
NSUB = 16
RPT = 640
NPAD = NSUB * RPT
CHUNK = 4000
NVREG = CHUNK // 16
NCHUNK = E // CHUNK
GP = 64
FB = 32
BLK = 512
PKSH = 15
PKMASK = (1 << PKSH) - 1
TRASH = RPT << PKSH
NEG_INF = float("-inf")


def _precompute_body(x1_ref, x2_ref, w_ref, b_ref, p_ref, q_ref):
    wt = w_ref[0:D, :]
    wb = w_ref[D : 2 * D, :]
    x1 = x1_ref[...]
    x2 = x2_ref[...]
    p_ref[...] = (
        jnp.dot(x1, wt - wb, preferred_element_type=jnp.float32) + b_ref[...]
    )
    q_ref[0] = jnp.dot(x1, wb, preferred_element_type=jnp.float32)
    q_ref[1] = jnp.dot(x2, wb, preferred_element_type=jnp.float32)


_precompute = pl.pallas_call(
    _precompute_body,
    grid=(NPAD // BLK,),
    in_specs=[
        pl.BlockSpec((BLK, D), lambda i: (i, 0)),
        pl.BlockSpec((BLK, D), lambda i: (i, 0)),
        pl.BlockSpec((2 * D, D), lambda i: (0, 0)),
        pl.BlockSpec((1, D), lambda i: (0, 0)),
    ],
    out_specs=[
        pl.BlockSpec((BLK, D), lambda i: (i, 0)),
        pl.BlockSpec((2, BLK, D), lambda i: (0, i, 0)),
    ],
    out_shape=[
        jax.ShapeDtypeStruct((NPAD, D), jnp.float32),
        jax.ShapeDtypeStruct((2, NPAD, D), jnp.float32),
    ],
)


def _sc_body(ei_ref, q_ref, p_ref, out_ref,
             rowbuf, colbuf, spk, scol, g_buf, f_buf, m_buf, smem,
             sem_r0, sem_r1, sem_c0, sem_c1, sem_g0, sem_g1):
    c = lax.axis_index("c")
    s = lax.axis_index("s")
    lo = s * RPT

    lo_v = jnp.broadcast_to(lo, (16,)).astype(jnp.int32)
    qb_v = jnp.broadcast_to(c * NPAD, (16,)).astype(jnp.int32)
    trash_v = jnp.full((16,), TRASH, jnp.int32)
    lsplats = [jnp.full((16,), l, jnp.int32) for l in range(16)]
    rbase = 2 * c * E
    cbase = (2 * c + 1) * E

    def start_io(ch, half, sem_r, sem_c):
        off = ch * CHUNK
        pltpu.async_copy(
            ei_ref.at[pl.ds(rbase + off, CHUNK)],
            rowbuf.at[pl.ds(half * CHUNK, CHUNK)], sem_r,
        )
        pltpu.async_copy(
            ei_ref.at[pl.ds(cbase + off, CHUNK)],
            colbuf.at[pl.ds(half * CHUNK, CHUNK)], sem_c,
        )

    def wait_io(ch, half, sem_r, sem_c):
        off = ch * CHUNK
        pltpu.make_async_copy(
            ei_ref.at[pl.ds(rbase + off, CHUNK)],
            rowbuf.at[pl.ds(half * CHUNK, CHUNK)], sem_r,
        ).wait()
        pltpu.make_async_copy(
            ei_ref.at[pl.ds(cbase + off, CHUNK)],
            colbuf.at[pl.ds(half * CHUNK, CHUNK)], sem_c,
        ).wait()

    def start_gather(p, half, sem_g):
        pltpu.async_copy(
            q_ref.at[scol.at[pl.ds(p * GP, GP)]],
            g_buf.at[pl.ds(half * GP, GP)], sem_g,
        )

    def wait_gather(p, half, sem_g):
        pltpu.make_async_copy(
            q_ref.at[scol.at[pl.ds(p * GP, GP)]],
            g_buf.at[pl.ds(half * GP, GP)], sem_g,
        ).wait()

    def init_m(i, _):
        for k in range(D // 16):
            m_buf[i, pl.ds(k * 16, 16)] = jnp.full((16,), NEG_INF, jnp.float32)
        return 0

    lax.fori_loop(0, RPT + 1, init_m, 0)

    start_io(0, 0, sem_r0, sem_c0)

    def do_chunk(ch, _):
        par = ch & 1
        base = par * CHUNK

        @pl.when(par == 0)
        def _():
            wait_io(ch, 0, sem_r0, sem_c0)

            @pl.when(ch + 1 < NCHUNK)
            def _():
                start_io(ch + 1, 1, sem_r1, sem_c1)

        @pl.when(par == 1)
        def _():
            wait_io(ch, 1, sem_r1, sem_c1)

            @pl.when(ch + 1 < NCHUNK)
            def _():
                start_io(ch + 1, 0, sem_r0, sem_c0)

        smem[0] = 0

        def scan_v(v, _):
            r = rowbuf[pl.ds(base + v * 16, 16)]
            cl = colbuf[pl.ds(base + v * 16, 16)]
            pk = ((r - lo_v) << PKSH) | (cl + qb_v)
            for l in range(16):
                dl = r[l] - lo
                hit = dl.astype(jnp.uint32) < jnp.uint32(RPT)

                @pl.when(hit)
                def _():
                    cnt = smem[0]
                    spk[pl.ds(cnt, 16)] = jnp.take(pk, lsplats[l])
                    smem[0] = cnt + 1

            return 0

        lax.fori_loop(0, NVREG, scan_v, 0)
        n_match = smem[0]

        for t in range(GP // 16 * 2):
            spk[pl.ds(n_match + t * 16, 16)] = trash_v

        def unpack(u, _):
            pkv = spk[pl.ds(u * 16, 16)]
            scol[pl.ds(u * 16, 16)] = pkv & PKMASK
            return 0

        lax.fori_loop(0, (n_match + 15) // 16 + GP // 16, unpack, 0)

        npieces = (n_match + GP - 1) // GP

        @pl.when(npieces > 0)
        def _():
            start_gather(0, 0, sem_g0)

        def do_piece(p, _):
            gpar = p & 1

            @pl.when(gpar == 0)
            def _():
                wait_gather(p, 0, sem_g0)

                @pl.when(p + 1 < npieces)
                def _():
                    start_gather(p + 1, 1, sem_g1)

            @pl.when(gpar == 1)
            def _():
                wait_gather(p, 1, sem_g1)

                @pl.when(p + 1 < npieces)
                def _():
                    start_gather(p + 1, 0, sem_g0)

            gb = gpar * GP
            for t in range(GP // 16):
                w = spk[pl.ds(p * GP + t * 16, 16)]
                for l in range(16):
                    d = w[l] >> PKSH
                    for k in range(D // 16):
                        sl = pl.ds(k * 16, 16)
                        m_buf[d, sl] = jnp.maximum(
                            m_buf[d, sl], g_buf[gb + t * 16 + l, sl]
                        )
            return 0

        lax.fori_loop(0, npieces, do_piece, 0)
        return 0

    lax.fori_loop(0, NCHUNK, do_chunk, 0)

    def fin(t, _):
        base = lo + t * FB
        pltpu.sync_copy(p_ref.at[pl.ds(base, FB)], f_buf)

        def fr(r, _):
            for k in range(D // 16):
                sl = pl.ds(k * 16, 16)
                f_buf[r, sl] = jnp.maximum(
                    f_buf[r, sl] + m_buf[t * FB + r, sl], 0.0
                )
            return 0

        lax.fori_loop(0, FB, fr, 0)
        pltpu.sync_copy(f_buf, out_ref.at[pl.ds(c * NPAD + base, FB)])
        return 0

    lax.fori_loop(0, RPT // FB, fin, 0)


_sc_run = pl.kernel(
    _sc_body,
    out_type=jax.ShapeDtypeStruct((2 * NPAD, D), jnp.float32),
    mesh=plsc.VectorSubcoreMesh(core_axis_name="c", subcore_axis_name="s"),
    scratch_types=[
        pltpu.VMEM((2 * CHUNK,), jnp.int32),
        pltpu.VMEM((2 * CHUNK,), jnp.int32),
        pltpu.VMEM((CHUNK + 2 * GP,), jnp.int32),
        pltpu.VMEM((CHUNK + 2 * GP,), jnp.int32),
        pltpu.VMEM((2 * GP, D), jnp.float32),
        pltpu.VMEM((FB, D), jnp.float32),
        pltpu.VMEM((RPT + 1, D), jnp.float32),
        pltpu.SMEM((8,), jnp.int32),
        pltpu.SemaphoreType.DMA,
        pltpu.SemaphoreType.DMA,
        pltpu.SemaphoreType.DMA,
        pltpu.SemaphoreType.DMA,
        pltpu.SemaphoreType.DMA,
        pltpu.SemaphoreType.DMA,
    ],
)


@jax.jit
def kernel(x1, edge_index1, x2, edge_index2, W, b):
    x1p = jnp.pad(x1, ((0, NPAD - N), (0, 0)))
    x2p = jnp.pad(x2, ((0, NPAD - N), (0, 0)))
    p, qs = _precompute(x1p, x2p, W, b.reshape(1, D))
    qflat = qs.reshape(2 * NPAD, D)
    ei4 = jnp.concatenate([edge_index1, edge_index2], axis=0).reshape(4 * E)
    outs = _sc_run(ei4, qflat, p)
    return jnp.concatenate([outs[:N], outs[NPAD : NPAD + N]], axis=-1)


# speedup vs baseline: 2.2774x; 1.6405x over previous
import jax
import jax.numpy as jnp
from jax import lax
from jax.experimental import pallas as pl
from jax.experimental.pallas import tpu as pltpu
from jax.experimental.pallas import tpu_sc as plsc

N = 10000
E = 320000
D = 128
NSUB = 16
RPT = 640
NPAD = NSUB * RPT
SCH = 512
SCH2 = SCH + 16
RBATCH = NSUB * SCH
EPAD = 327680
NROUND = EPAD // RBATCH
SEG = NSUB * SCH
XBLK = NSUB * SCH2
GP = 32
FB = 32
BLK = 512
EBLK = 4096
NEG_INF = float("-inf")


def _precompute_body(x1_ref, x2_ref, w_ref, b_ref, p_ref, q_ref):
    wt = w_ref[0:D, :]
    wb = w_ref[D : 2 * D, :]
    x1 = x1_ref[...]
    x2 = x2_ref[...]
    p_ref[...] = (
        jnp.dot(x1, wt - wb, preferred_element_type=jnp.float32) + b_ref[...]
    )
    q_ref[0] = jnp.dot(x1, wb, preferred_element_type=jnp.float32)
    q_ref[1] = jnp.dot(x2, wb, preferred_element_type=jnp.float32)


_precompute = pl.pallas_call(
    _precompute_body,
    grid=(NPAD // BLK,),
    in_specs=[
        pl.BlockSpec((BLK, D), lambda i: (i, 0)),
        pl.BlockSpec((BLK, D), lambda i: (i, 0)),
        pl.BlockSpec((2 * D, D), lambda i: (0, 0)),
        pl.BlockSpec((1, D), lambda i: (0, 0)),
    ],
    out_specs=[
        pl.BlockSpec((BLK, D), lambda i: (i, 0)),
        pl.BlockSpec((2, BLK, D), lambda i: (0, i, 0)),
    ],
    out_shape=[
        jax.ShapeDtypeStruct((NPAD, D), jnp.float32),
        jax.ShapeDtypeStruct((2, NPAD, D), jnp.float32),
    ],
)


def _pack_body(row_ref, col_ref, o_ref):
    o_ref[0] = (row_ref[0] << 16) | col_ref[0]


_EROW = EPAD // 128

_pack = pl.pallas_call(
    _pack_body,
    grid=(2,),
    in_specs=[
        pl.BlockSpec((1, _EROW, 128), lambda c: (2 * c, 0, 0)),
        pl.BlockSpec((1, _EROW, 128), lambda c: (2 * c + 1, 0, 0)),
    ],
    out_specs=pl.BlockSpec((1, _EROW, 128), lambda c: (c, 0, 0)),
    out_shape=jax.ShapeDtypeStruct((2, _EROW, 128), jnp.int32),
)


def _sc_body(ep_ref, q_ref, p_ref, out_ref, exch_ref,
             ebuf, bkts, stag, spk, scol, g_buf, f_buf, m_buf, smem,
             sem_e0, sem_e1, sem_x, sem_g0, sem_g1):
    c = lax.axis_index("c")
    s = lax.axis_index("s")
    lo = s * RPT

    qb_v = jnp.broadcast_to(c * NPAD, (16,)).astype(jnp.int32)
    trash_v = jnp.broadcast_to((lo + RPT) << 16, (16,)).astype(jnp.int32)
    lsplats = [jnp.full((16,), l, jnp.int32) for l in range(16)]
    ebase = c * EPAD + s * SCH

    def start_edges(r, half, sem):
        pltpu.async_copy(
            ep_ref.at[pl.ds(ebase + r * RBATCH, SCH)],
            ebuf.at[pl.ds(half * SCH, SCH)], sem,
        )

    def wait_edges(r, half, sem):
        pltpu.make_async_copy(
            ep_ref.at[pl.ds(ebase + r * RBATCH, SCH)],
            ebuf.at[pl.ds(half * SCH, SCH)], sem,
        ).wait()

    def start_gather(p, half, sem):
        pltpu.async_copy(
            q_ref.at[scol.at[pl.ds(p * GP, GP)]],
            g_buf.at[pl.ds(half * GP, GP)], sem,
        )

    def wait_gather(p, half, sem):
        pltpu.make_async_copy(
            q_ref.at[scol.at[pl.ds(p * GP, GP)]],
            g_buf.at[pl.ds(half * GP, GP)], sem,
        ).wait()

    def init_m(i, _):
        for k in range(D // 16):
            m_buf[i, pl.ds(k * 16, 16)] = jnp.full((16,), NEG_INF, jnp.float32)
        return 0

    lax.fori_loop(0, RPT + 1, init_m, 0)

    start_edges(0, 0, sem_e0)

    def do_round(r, _):
        par = r & 1
        base = par * SCH

        @pl.when(par == 0)
        def _():
            wait_edges(r, 0, sem_e0)

            @pl.when(r + 1 < NROUND)
            def _():
                start_edges(r + 1, 1, sem_e1)

        @pl.when(par == 1)
        def _():
            wait_edges(r, 1, sem_e1)

            @pl.when(r + 1 < NROUND)
            def _():
                start_edges(r + 1, 0, sem_e0)

        for b in range(NSUB):
            smem[b] = 0

        def scan_v(v, _):
            w = ebuf[pl.ds(base + v * 16, 16)]
            for l in range(16):
                wl = w[l]
                rl = wl >> 16
                b = ((rl >> 7) * 52429) >> 18
                cb = smem[b]
                bkts[pl.ds(b * SCH2 + 16 + cb, 16)] = jnp.take(w, lsplats[l])
                smem[b] = cb + 1

            return 0

        lax.fori_loop(0, SCH // 16, scan_v, 0)

        for b in range(NSUB):
            bkts[pl.ds(b * SCH2, 16)] = jnp.broadcast_to(
                smem[b], (16,)
            ).astype(jnp.int32)
        for half in range(2):
            for b in range(half * 8, half * 8 + 8):
                pltpu.async_copy(
                    bkts.at[pl.ds(b * SCH2, SCH2)],
                    exch_ref.at[pl.ds((c * NSUB + b) * XBLK + s * SCH2, SCH2)],
                    sem_x,
                )
            for b in range(half * 8, half * 8 + 8):
                pltpu.make_async_copy(
                    bkts.at[pl.ds(b * SCH2, SCH2)],
                    exch_ref.at[pl.ds((c * NSUB + b) * XBLK + s * SCH2, SCH2)],
                    sem_x,
                ).wait()
        plsc.subcore_barrier()

        pltpu.sync_copy(
            exch_ref.at[pl.ds((c * NSUB + s) * XBLK, XBLK)], stag
        )
        plsc.subcore_barrier()

        tot = jnp.int32(0)
        for w in range(NSUB):
            cw = stag[pl.ds(w * SCH2, 16)][0]
            base_w = tot

            def cpy(k, _):
                spk[pl.ds(base_w + k * 16, 16)] = stag[
                    pl.ds(w * SCH2 + 16 + k * 16, 16)
                ]
                return 0

            lax.fori_loop(0, (cw + 15) // 16, cpy, 0)
            tot = tot + cw

        n_match = tot
        for t in range(GP // 16 * 2):
            spk[pl.ds(n_match + t * 16, 16)] = trash_v

        def unpack(u, _):
            pkv = spk[pl.ds(u * 16, 16)]
            scol[pl.ds(u * 16, 16)] = (pkv & 0xFFFF) + qb_v
            return 0

        lax.fori_loop(0, (n_match + 15) // 16 + GP // 16, unpack, 0)

        npieces = (n_match + GP - 1) // GP

        @pl.when(npieces > 0)
        def _():
            start_gather(0, 0, sem_g0)

        def do_piece(p, _):
            gpar = p & 1

            @pl.when(gpar == 0)
            def _():
                wait_gather(p, 0, sem_g0)

                @pl.when(p + 1 < npieces)
                def _():
                    start_gather(p + 1, 1, sem_g1)

            @pl.when(gpar == 1)
            def _():
                wait_gather(p, 1, sem_g1)

                @pl.when(p + 1 < npieces)
                def _():
                    start_gather(p + 1, 0, sem_g0)

            gb = gpar * GP
            for t in range(GP // 16):
                wv = spk[pl.ds(p * GP + t * 16, 16)]
                for l in range(16):
                    d = (wv[l] >> 16) - lo
                    for k in range(D // 16):
                        sl = pl.ds(k * 16, 16)
                        m_buf[d, sl] = jnp.maximum(
                            m_buf[d, sl], g_buf[gb + t * 16 + l, sl]
                        )
            return 0

        lax.fori_loop(0, npieces, do_piece, 0)
        return 0

    lax.fori_loop(0, NROUND, do_round, 0)

    def fin(t, _):
        fb = lo + t * FB
        pltpu.sync_copy(p_ref.at[pl.ds(fb, FB)], f_buf)

        def fr(rr, _):
            for k in range(D // 16):
                sl = pl.ds(k * 16, 16)
                f_buf[rr, sl] = jnp.maximum(
                    f_buf[rr, sl] + m_buf[t * FB + rr, sl], 0.0
                )
            return 0

        lax.fori_loop(0, FB, fr, 0)
        pltpu.sync_copy(f_buf, out_ref.at[pl.ds(c * NPAD + fb, FB)])
        return 0

    lax.fori_loop(0, RPT // FB, fin, 0)


_sc_run = pl.kernel(
    _sc_body,
    out_type=[
        jax.ShapeDtypeStruct((2 * NPAD, D), jnp.float32),
        jax.ShapeDtypeStruct((2 * NSUB * XBLK,), jnp.int32),
    ],
    mesh=plsc.VectorSubcoreMesh(core_axis_name="c", subcore_axis_name="s"),
    scratch_types=[
        pltpu.VMEM((2 * SCH,), jnp.int32),
        pltpu.VMEM((NSUB * SCH2,), jnp.int32),
        pltpu.VMEM((NSUB * SCH2,), jnp.int32),
        pltpu.VMEM((SEG + 2 * GP,), jnp.int32),
        pltpu.VMEM((SEG + 2 * GP,), jnp.int32),
        pltpu.VMEM((2 * GP, D), jnp.float32),
        pltpu.VMEM((FB, D), jnp.float32),
        pltpu.VMEM((RPT + 1, D), jnp.float32),
        pltpu.SMEM((32,), jnp.int32),
        pltpu.SemaphoreType.DMA,
        pltpu.SemaphoreType.DMA,
        pltpu.SemaphoreType.DMA,
        pltpu.SemaphoreType.DMA,
        pltpu.SemaphoreType.DMA,
    ],
)


@jax.jit
def kernel(x1, edge_index1, x2, edge_index2, W, b):
    x1p = jnp.pad(x1, ((0, NPAD - N), (0, 0)))
    x2p = jnp.pad(x2, ((0, NPAD - N), (0, 0)))
    p, qs = _precompute(x1p, x2p, W, b.reshape(1, D))
    qflat = jnp.concatenate(
        [qs.reshape(2 * NPAD, D), jnp.full((16, D), NEG_INF, jnp.float32)]
    )
    padr = (jnp.arange(EPAD - E, dtype=jnp.int32) % NSUB) * RPT
    padc1 = jnp.full((EPAD - E,), 2 * NPAD, jnp.int32)
    padc2 = jnp.full((EPAD - E,), NPAD, jnp.int32)
    ei = jnp.stack(
        [
            jnp.concatenate([edge_index1[0], padr]),
            jnp.concatenate([edge_index1[1], padc1]),
            jnp.concatenate([edge_index2[0], padr]),
            jnp.concatenate([edge_index2[1], padc2]),
        ]
    ).reshape(4, _EROW, 128)
    epk = _pack(ei, ei).reshape(2 * EPAD)
    outs, _ = _sc_run(epk, qflat, p)
    return jnp.concatenate([outs[:N], outs[NPAD : NPAD + N]], axis=-1)
